# Initial kernel scaffold; baseline (speedup 1.0000x reference)
import jax
import jax.numpy as jnp
from jax.experimental import pallas as pl


def kernel(zeo, syn, smis_seq, char_embed, type_embed, pe):
    raise NotImplementedError("write your pallas kernel here")


# speedup vs baseline: 6.0395x; 6.0395x over previous
import functools

import jax
import jax.numpy as jnp
from jax import lax
from jax.experimental import pallas as pl
from jax.experimental.pallas import tpu as pltpu
from jax.experimental.pallas import tpu_sc as plsc

_B, _T, _D = 4096, 125, 128
_NC, _NS = 2, 16
_NW = _NC * _NS
_BPW = _B // _NW
_LANES = 16
_DV = _D // _LANES

_mesh = plsc.VectorSubcoreMesh(
    core_axis_name="c", subcore_axis_name="s", num_cores=_NC, num_subcores=_NS
)


@functools.partial(
    pl.kernel,
    out_type=(
        jax.ShapeDtypeStruct((_B, _T, _D), jnp.float32),
        jax.ShapeDtypeStruct((_B, 1, _D), jnp.float32),
        jax.ShapeDtypeStruct((_B, 1, _D), jnp.float32),
    ),
    mesh=_mesh,
    scratch_types=[
        pltpu.VMEM((_BPW, _T), jnp.int32),
        pltpu.VMEM((_T, _D), jnp.float32),
        pltpu.VMEM((3, _D), jnp.float32),
        pltpu.VMEM((_T, _D), jnp.float32),
        pltpu.VMEM((_BPW, 1, _D), jnp.float32),
        pltpu.SemaphoreType.DMA,
    ],
)
def _embed_sc(zeo, syn, idx_hbm, table, te_hbm, pe_hbm,
              out_seq, out_zeo, out_syn,
              idx_v, bias_v, te_v, rows_v, zs_v, sem):
    wid = lax.axis_index("s") * _NC + lax.axis_index("c")
    base = wid * _BPW

    pltpu.sync_copy(te_hbm, te_v)
    pltpu.sync_copy(pe_hbm, bias_v)
    pltpu.sync_copy(idx_hbm.at[pl.ds(base, _BPW)], idx_v)

    def bias_body(t, carry):
        for d in range(_DV):
            sl = pl.ds(d * _LANES, _LANES)
            bias_v[t, sl] = bias_v[t, sl] + te_v[2, sl]
        return carry
    lax.fori_loop(0, _T, bias_body, 0)

    pltpu.sync_copy(zeo.at[pl.ds(base, _BPW)], zs_v)
    def zeo_body(i, carry):
        for d in range(_DV):
            sl = pl.ds(d * _LANES, _LANES)
            zs_v[i, 0, sl] = zs_v[i, 0, sl] + te_v[0, sl]
        return carry
    lax.fori_loop(0, _BPW, zeo_body, 0)
    pltpu.sync_copy(zs_v, out_zeo.at[pl.ds(base, _BPW)])

    pltpu.sync_copy(syn.at[pl.ds(base, _BPW)], zs_v)
    def syn_body(i, carry):
        for d in range(_DV):
            sl = pl.ds(d * _LANES, _LANES)
            zs_v[i, 0, sl] = zs_v[i, 0, sl] + te_v[1, sl]
        return carry
    lax.fori_loop(0, _BPW, syn_body, 0)
    pltpu.sync_copy(zs_v, out_syn.at[pl.ds(base, _BPW)])

    def gather_body(i, carry):
        pltpu.async_copy(table.at[idx_v.at[i]], rows_v, sem).wait()
        def add_body(t, c2):
            for d in range(_DV):
                sl = pl.ds(d * _LANES, _LANES)
                rows_v[t, sl] = rows_v[t, sl] + bias_v[t, sl]
            return c2
        lax.fori_loop(0, _T, add_body, 0)
        pltpu.sync_copy(rows_v, out_seq.at[base + i])
        return carry
    lax.fori_loop(0, _BPW, gather_body, 0)


def kernel(zeo, syn, smis_seq, char_embed, type_embed, pe):
    idx = smis_seq.astype(jnp.int32)
    pe2d = pe.reshape(_T, _D)
    return _embed_sc(zeo, syn, idx, char_embed, type_embed, pe2d)


# speedup vs baseline: 7.9783x; 1.3210x over previous
import functools

import jax
import jax.numpy as jnp
from jax import lax
from jax.experimental import pallas as pl
from jax.experimental.pallas import tpu as pltpu
from jax.experimental.pallas import tpu_sc as plsc

_B, _T, _D = 4096, 125, 128
_NC, _NS = 2, 16
_NW = _NC * _NS
_BPW = _B // _NW
_LANES = 16
_DV = _D // _LANES
_NBUF = 4

_mesh = plsc.VectorSubcoreMesh(
    core_axis_name="c", subcore_axis_name="s", num_cores=_NC, num_subcores=_NS
)


@functools.partial(
    pl.kernel,
    out_type=(
        jax.ShapeDtypeStruct((_B, _T, _D), jnp.float32),
        jax.ShapeDtypeStruct((_B, 1, _D), jnp.float32),
        jax.ShapeDtypeStruct((_B, 1, _D), jnp.float32),
    ),
    mesh=_mesh,
    scratch_types=[
        pltpu.VMEM((_BPW, _T), jnp.int32),
        pltpu.VMEM((_T, _D), jnp.float32),
        pltpu.VMEM((3, _D), jnp.float32),
        [pltpu.VMEM((_T, _D), jnp.float32)] * _NBUF,
        pltpu.VMEM((_BPW, 1, _D), jnp.float32),
        [pltpu.SemaphoreType.DMA] * _NBUF,
        [pltpu.SemaphoreType.DMA] * _NBUF,
    ],
)
def _embed_sc(zeo, syn, idx_hbm, table, te_hbm, pe_hbm,
              out_seq, out_zeo, out_syn,
              idx_v, bias_v, te_v, rows, zs_v, gsem, osem):
    wid = lax.axis_index("s") * _NC + lax.axis_index("c")
    base = wid * _BPW

    pltpu.sync_copy(te_hbm, te_v)
    pltpu.sync_copy(pe_hbm, bias_v)
    pltpu.sync_copy(idx_hbm.at[pl.ds(base, _BPW)], idx_v)

    def bias_body(t, carry):
        for d in range(_DV):
            sl = pl.ds(d * _LANES, _LANES)
            plsc.addupdate(bias_v.at[t, sl], te_v[2, sl])
        return carry
    lax.fori_loop(0, _T, bias_body, 0)

    for src, dst, row in ((zeo, out_zeo, 0), (syn, out_syn, 1)):
        pltpu.sync_copy(src.at[pl.ds(base, _BPW)], zs_v)
        def zs_body(i, carry, row=row):
            for d in range(_DV):
                sl = pl.ds(d * _LANES, _LANES)
                plsc.addupdate(zs_v.at[i, 0, sl], te_v[row, sl])
            return carry
        lax.fori_loop(0, _BPW, zs_body, 0)
        pltpu.sync_copy(zs_v, dst.at[pl.ds(base, _BPW)])

    def g_copy(k, j):
        return pltpu.make_async_copy(table.at[idx_v.at[k]], rows[j], gsem[j])

    def o_copy(k, j):
        return pltpu.make_async_copy(rows[j], out_seq.at[base + k], osem[j])

    def add_bias(k, j):
        def add_body(t5, carry):
            for u in range(5):
                t = t5 * 5 + u
                for d in range(_DV):
                    sl = pl.ds(d * _LANES, _LANES)
                    plsc.addupdate(rows[j].at[t, sl], bias_v[t, sl])
            return carry
        lax.fori_loop(0, _T // 5, add_body, 0)

    g_copy(0, 0).start()
    g_copy(1, 1).start()
    for k in (0, 1):
        g_copy(k, k).wait()
        add_bias(k, k)
        o_copy(k, k).start()
        g_copy(k + 2, k + 2).start()

    def main_body(k4, carry):
        for j in range(_NBUF):
            k = 2 + k4 * _NBUF + j
            buf = (2 + j) % _NBUF
            nbuf = j % _NBUF
            g_copy(k, buf).wait()
            add_bias(k, buf)
            o_copy(k, buf).start()
            o_copy(k - 2, nbuf).wait()
            g_copy(k + 2, nbuf).start()
        return carry
    lax.fori_loop(0, (_BPW - _NBUF) // _NBUF, main_body, 0)

    for k in (_BPW - 2, _BPW - 1):
        j = k % _NBUF
        g_copy(k, j).wait()
        add_bias(k, j)
        o_copy(k, j).start()
    for k in range(_BPW - _NBUF, _BPW):
        o_copy(k, k % _NBUF).wait()


def kernel(zeo, syn, smis_seq, char_embed, type_embed, pe):
    idx = smis_seq.astype(jnp.int32)
    pe2d = pe.reshape(_T, _D)
    return _embed_sc(zeo, syn, idx, char_embed, type_embed, pe2d)
